# Initial kernel scaffold; baseline (speedup 1.0000x reference)
import jax
import jax.numpy as jnp
from jax.experimental import pallas as pl


def kernel(edge_index_mp, emb, weight, p, W_ih, W_hh, b_ih, b_hh):
    raise NotImplementedError("write your pallas kernel here")


# speedup vs baseline: 4.6275x; 4.6275x over previous
import functools

import jax
import jax.numpy as jnp
from jax import lax
from jax.experimental import pallas as pl
from jax.experimental.pallas import tpu as pltpu
from jax.experimental.pallas import tpu_sc as plsc

N = 10000
D = 256
E = 160000
NPAD = 10240
K = 256
SURV = 1024
ZROW = 10000
HALF = 5000
HROWS = 5120

_I32_MIN = -2147483648


def _f32_key(x):
    b = lax.bitcast_convert_type(x, jnp.int32)
    return lax.bitwise_xor(b, lax.bitwise_and(lax.shift_right_arithmetic(b, 31),
                                              jnp.int32(0x7FFFFFFF)))


def _score_body(x_ref, p_ref, score_ref, keys_ref, thr_ref):
    p = p_ref[...]
    rn = lax.rsqrt(jnp.sum(p * p))
    score = lax.dot_general(p, x_ref[...],
                            (((1,), (1,)), ((), ())),
                            preferred_element_type=jnp.float32) * rn
    score_ref[...] = score
    key = _f32_key(score)
    keys_ref[...] = key
    ukey = lax.bitcast_convert_type(
        lax.bitwise_xor(key, jnp.int32(_I32_MIN)), jnp.uint32)

    def bit_step(i, t):
        cand = lax.bitwise_or(t, lax.shift_left(jnp.uint32(1),
                                                jnp.uint32(31) - i.astype(jnp.uint32)))
        cnt = jnp.sum((ukey >= cand).astype(jnp.int32))
        return jnp.where(cnt >= K, cand, t)

    t_u = lax.fori_loop(0, 32, bit_step, jnp.uint32(0))
    t_i = lax.bitwise_xor(lax.bitcast_convert_type(t_u, jnp.int32), jnp.int32(_I32_MIN))
    thr_ref[...] = jnp.full((1, 16), t_i, jnp.int32)


def _k_score(x, p):
    return pl.pallas_call(
        _score_body,
        out_shape=(jax.ShapeDtypeStruct((1, N), jnp.float32),
                   jax.ShapeDtypeStruct((1, N), jnp.int32),
                   jax.ShapeDtypeStruct((1, 16), jnp.int32)),
        name="egcn_score",
    )(x, p.reshape(1, D))


def _compact_body(keys_hbm, score_hbm, thr_hbm, skey_hbm, sidx_hbm, sval_hbm,
                  kbuf, sbuf, tbuf, okey, oidx, oval):
    wid = lax.axis_index("s") * 2 + lax.axis_index("c")

    @pl.when(wid == 0)
    def _():
        pltpu.sync_copy(keys_hbm, kbuf)
        pltpu.sync_copy(score_hbm, sbuf)
        pltpu.sync_copy(thr_hbm, tbuf)
        tvec = tbuf[...]
        sent_k = jnp.full((16,), jnp.int32(_I32_MIN), jnp.int32)
        sent_i = jnp.full((16,), jnp.int32(0), jnp.int32)
        sent_v = jnp.full((16,), 0.0, jnp.float32)

        def zero_step(i, _):
            okey[pl.ds(i * 16, 16)] = sent_k
            oidx[pl.ds(i * 16, 16)] = sent_i
            oval[pl.ds(i * 16, 16)] = sent_v
            return 0

        lax.fori_loop(0, SURV // 16, zero_step, 0)
        lane = lax.iota(jnp.int32, 16)

        def step(i, off):
            kv = kbuf[pl.ds(i * 16, 16)]
            sv = sbuf[pl.ds(i * 16, 16)]
            m = kv >= tvec
            pos = off + plsc.cumsum(m.astype(jnp.int32)) - 1
            m = jnp.logical_and(m, pos < SURV)
            plsc.store_scatter(okey, [pos], kv, mask=m)
            plsc.store_scatter(oidx, [pos], i * 16 + lane, mask=m)
            plsc.store_scatter(oval, [pos], sv, mask=m)
            return off + plsc.all_reduce_population_count(m)

        lax.fori_loop(0, N // 16, step, jnp.zeros((16,), jnp.int32))
        pltpu.sync_copy(okey, skey_hbm)
        pltpu.sync_copy(oidx, sidx_hbm)
        pltpu.sync_copy(oval, sval_hbm)


def _k_compact(keys, score, thr):
    mesh = plsc.VectorSubcoreMesh(core_axis_name="c", subcore_axis_name="s")
    f = pl.kernel(
        _compact_body,
        out_type=(jax.ShapeDtypeStruct((SURV,), jnp.int32),
                  jax.ShapeDtypeStruct((SURV,), jnp.int32),
                  jax.ShapeDtypeStruct((SURV,), jnp.float32)),
        mesh=mesh,
        scratch_types=[pltpu.VMEM((N,), jnp.int32),
                       pltpu.VMEM((N,), jnp.float32),
                       pltpu.VMEM((16,), jnp.int32),
                       pltpu.VMEM((SURV,), jnp.int32),
                       pltpu.VMEM((SURV,), jnp.int32),
                       pltpu.VMEM((SURV,), jnp.float32)],
        compiler_params=pltpu.CompilerParams(needs_layout_passes=False),
        name="egcn_compact",
    )
    return f(keys.reshape(N), score.reshape(N), thr.reshape(16))


def _rank_body(kv_ref, iv_ref, ks_ref, is_ref, rank_ref):
    keys = kv_ref[...]
    idxs = iv_ref[...]

    def step(j, acc):
        r = j // 128
        c = lax.rem(j, 128)
        kj = ks_ref[r, c]
        ij = is_ref[r, c]
        gt = (kj > keys).astype(jnp.int32)
        eq = jnp.logical_and(kj == keys, ij < idxs).astype(jnp.int32)
        return acc + gt + eq

    rank_ref[...] = lax.fori_loop(0, SURV, step,
                                  jnp.zeros((8, 128), jnp.int32))


def _k_rank(skey, sidx):
    k2 = skey.reshape(8, 128)
    i2 = sidx.reshape(8, 128)
    return pl.pallas_call(
        _rank_body,
        out_shape=jax.ShapeDtypeStruct((8, 128), jnp.int32),
        in_specs=[pl.BlockSpec((8, 128), lambda: (0, 0)),
                  pl.BlockSpec((8, 128), lambda: (0, 0)),
                  pl.BlockSpec(memory_space=pltpu.SMEM),
                  pl.BlockSpec(memory_space=pltpu.SMEM)],
        name="egcn_rank",
    )(k2, i2, k2, i2)


def _gather_body(sidx_hbm, sval_hbm, rank_hbm, x_hbm, xg_hbm, topv_hbm,
                 ibuf, vbuf, rbuf, topi, topv, rows, sem):
    wid = lax.axis_index("s") * 2 + lax.axis_index("c")

    @pl.when(wid == 0)
    def _():
        pltpu.sync_copy(sidx_hbm, ibuf)
        pltpu.sync_copy(sval_hbm, vbuf)
        pltpu.sync_copy(rank_hbm, rbuf)

        def step(i, _):
            rk = rbuf[pl.ds(i * 16, 16)]
            m = rk < K
            plsc.store_scatter(topi, [rk], ibuf[pl.ds(i * 16, 16)], mask=m)
            plsc.store_scatter(topv, [rk], vbuf[pl.ds(i * 16, 16)], mask=m)
            return 0

        lax.fori_loop(0, SURV // 16, step, 0)
        pltpu.async_copy(x_hbm.at[topi], rows, sem).wait()
        pltpu.sync_copy(rows, xg_hbm)
        pltpu.sync_copy(topv, topv_hbm)


def _k_gather(sidx, sval, rank, x):
    mesh = plsc.VectorSubcoreMesh(core_axis_name="c", subcore_axis_name="s")
    f = pl.kernel(
        _gather_body,
        out_type=(jax.ShapeDtypeStruct((K, D), jnp.float32),
                  jax.ShapeDtypeStruct((K,), jnp.float32)),
        mesh=mesh,
        scratch_types=[pltpu.VMEM((SURV,), jnp.int32),
                       pltpu.VMEM((SURV,), jnp.float32),
                       pltpu.VMEM((SURV,), jnp.int32),
                       pltpu.VMEM((K,), jnp.int32),
                       pltpu.VMEM((K,), jnp.float32),
                       pltpu.VMEM((K, D), jnp.float32),
                       pltpu.SemaphoreType.DMA],
        compiler_params=pltpu.CompilerParams(needs_layout_passes=False),
        name="egcn_gather",
    )
    return f(sidx, sval, rank.reshape(SURV), x)


def _gru_body(xg_ref, tv_ref, w_ref, wih_ref, whh_ref, bih_ref, bhh_ref,
              wnew_ref):
    xt = xg_ref[...] * jnp.tanh(tv_ref[...]).reshape(K, 1)
    gi = jnp.dot(xt, wih_ref[...], preferred_element_type=jnp.float32) \
        + bih_ref[...]
    gh = jnp.dot(w_ref[...], whh_ref[...], preferred_element_type=jnp.float32) \
        + bhh_ref[...]
    i_r, i_z, i_n = gi[:, :D], gi[:, D:2 * D], gi[:, 2 * D:]
    h_r, h_z, h_n = gh[:, :D], gh[:, D:2 * D], gh[:, 2 * D:]
    r = jax.nn.sigmoid(i_r + h_r)
    z = jax.nn.sigmoid(i_z + h_z)
    n = jnp.tanh(i_n + r * h_n)
    wnew_ref[...] = (1.0 - z) * n + z * w_ref[...]


def _k_gru(xg, topv, weight, w_ih_t, w_hh_t, b_ih, b_hh):
    return pl.pallas_call(
        _gru_body,
        out_shape=jax.ShapeDtypeStruct((D, D), jnp.float32),
        name="egcn_gru",
    )(xg, topv.reshape(1, K), weight, w_ih_t, w_hh_t,
      b_ih.reshape(1, 3 * D), b_hh.reshape(1, 3 * D))


EPT = N
DROWS = 79


def _deg_body(dst_hbm, deg_hbm, dbuf, idx2, ones2, zbuf, cbuf, degsp):
    c = lax.axis_index("c")
    s = lax.axis_index("s")

    @pl.when(c == 0)
    def _():
        def zstep(i, _):
            zbuf[pl.ds(i * 16, 16)] = jnp.zeros((16,), jnp.float32)
            return 0

        lax.fori_loop(0, 640 // 16, zstep, 0)
        pltpu.sync_copy(zbuf, degsp.at[pl.ds(s * 640, 640)])
        plsc.subcore_barrier()

        pltpu.sync_copy(dst_hbm.at[pl.ds(s * EPT, EPT)],
                        dbuf.at[pl.ds(0, EPT)])
        lane = lax.iota(jnp.int32, 16)

        def prep(i, _):
            r = i // 8
            col = lax.rem(i, 8) * 16
            pos = i * 16 + lane
            real = pos < EPT
            dv = dbuf[pl.ds(i * 16, 16)]
            dv = jnp.where(real, dv, lax.bitwise_and(pos, jnp.int32(8191)))
            idx2[r, pl.ds(col, 16)] = dv
            ones2[r, pl.ds(col, 16)] = jnp.where(real, 1.0,
                                                 0.0).astype(jnp.float32)
            return 0

        lax.fori_loop(0, DROWS * 8, prep, 0)

        def scat(j, _):
            pltpu.sync_copy(ones2.at[j], degsp.at[idx2.at[j]], add=True)
            return 0

        lax.fori_loop(0, DROWS, scat, 0)
        plsc.subcore_barrier()
        pltpu.sync_copy(degsp.at[pl.ds(s * 640, 640)], cbuf)
        pltpu.sync_copy(cbuf, deg_hbm.at[pl.ds(s * 640, 640)])


def _k_deg(dst):
    mesh = plsc.VectorSubcoreMesh(core_axis_name="c", subcore_axis_name="s")
    f = pl.kernel(
        _deg_body,
        out_type=jax.ShapeDtypeStruct((NPAD,), jnp.float32),
        mesh=mesh,
        scratch_types=[pltpu.VMEM((DROWS * 128,), jnp.int32),
                       pltpu.VMEM((DROWS, 128), jnp.int32),
                       pltpu.VMEM((DROWS, 128), jnp.float32),
                       pltpu.VMEM((640,), jnp.float32),
                       pltpu.VMEM((640,), jnp.float32),
                       pltpu.VMEM_SHARED((NPAD,), jnp.float32)],
        name="egcn_deg",
    )
    return f(dst)


def _dinv_body(deg_ref, dinv_ref):
    row = lax.broadcasted_iota(jnp.int32, (80, 128), 0)
    col = lax.broadcasted_iota(jnp.int32, (80, 128), 1)
    gid = row * 128 + col
    d = lax.rsqrt(deg_ref[...] + 1.0)
    dinv_ref[...] = jnp.where(gid < N, d, 0.0)


def _k_dinv(deg):
    return pl.pallas_call(
        _dinv_body,
        out_shape=jax.ShapeDtypeStruct((80, 128), jnp.float32),
        name="egcn_dinv",
    )(deg.reshape(80, 128))


def _xw_body(x_ref, w_ref, dv_ref, yw_ref):
    acc = jnp.dot(x_ref[...], w_ref[...], preferred_element_type=jnp.float32)
    yw_ref[...] = acc * dv_ref[...]


def _k_xw(xp, w_new, dinv):
    blk = 1024
    return pl.pallas_call(
        _xw_body,
        grid=(NPAD // blk,),
        in_specs=[pl.BlockSpec((blk, D), lambda i: (i, 0)),
                  pl.BlockSpec((D, D), lambda i: (0, 0)),
                  pl.BlockSpec((blk, 1), lambda i: (i, 0))],
        out_specs=pl.BlockSpec((blk, D), lambda i: (i, 0)),
        out_shape=jax.ShapeDtypeStruct((NPAD, D), jnp.float32),
        name="egcn_xw",
    )(xp, w_new, dinv.reshape(NPAD, 1))


OWN = 312
ACCR = 336
CH = 64
SEG = 4000
LROWS = 64
MAGIC = 107549


def _edges_body(src_hbm, dst_hbm, yw_hbm, acc_hbm,
                sseg, dseg, lsr, llc, rows, acc, gsem):
    c = lax.axis_index("c")
    s = lax.axis_index("s")
    w = s * 2 + c
    off = w * OWN
    lane = lax.iota(jnp.int32, 16)

    pltpu.sync_copy(yw_hbm.at[pl.ds(ZROW, 128)], acc.at[pl.ds(0, 128)])
    pltpu.sync_copy(yw_hbm.at[pl.ds(ZROW, 128)], acc.at[pl.ds(128, 128)])
    pltpu.sync_copy(yw_hbm.at[pl.ds(ZROW, 80)], acc.at[pl.ds(256, 80)])

    def seg_body(g, _unused):
        pltpu.sync_copy(src_hbm.at[pl.ds(g * SEG, SEG)], sseg)
        pltpu.sync_copy(dst_hbm.at[pl.ds(g * SEG, SEG)], dseg)

        def filt(i, cv):
            sv = sseg[pl.ds(i * 16, 16)]
            dv = dseg[pl.ds(i * 16, 16)]
            t = jnp.minimum(
                lax.shift_right_logical(dv * MAGIC, 25), 31)
            m = t == w
            p = cv + plsc.cumsum(m.astype(jnp.int32)) - 1
            rr = lax.shift_right_logical(p, 6)
            qq = lax.bitwise_and(p, 63)
            plsc.store_scatter(lsr, [rr, qq], sv, mask=m)
            plsc.store_scatter(llc, [rr, qq], dv - off, mask=m)
            return cv + plsc.all_reduce_population_count(m)

        cnt = lax.fori_loop(0, SEG // 16, filt, jnp.zeros((16,), jnp.int32))

        for j in range(4):
            pos = cnt + j * 16 + lane
            msk = pos < LROWS * CH
            rr = lax.shift_right_logical(pos, 6)
            qq = lax.bitwise_and(pos, 63)
            plsc.store_scatter(
                lsr, [rr, qq],
                ZROW + lax.bitwise_and(pos, jnp.int32(127)), mask=msk)
            plsc.store_scatter(
                llc, [rr, qq],
                328 + lax.bitwise_and(pos, jnp.int32(7)), mask=msk)

        nch = lax.shift_right_logical(cnt[0] + (CH - 1), 6)

        def drain(ch, _):
            pltpu.async_copy(yw_hbm.at[lsr.at[ch]], rows, gsem).wait()
            for j in range(CH // 16):
                lv = llc[ch, pl.ds(j * 16, 16)]
                for k in range(16):
                    rl = lv[k]
                    e = j * 16 + k
                    for cb in range(D // 16):
                        plsc.addupdate(acc.at[rl, pl.ds(cb * 16, 16)],
                                       rows[e, pl.ds(cb * 16, 16)])
            return 0

        lax.fori_loop(0, nch, drain, 0)
        return 0

    lax.fori_loop(0, E // SEG, seg_body, 0)

    @pl.when(w < 31)
    def _():
        pltpu.sync_copy(acc.at[pl.ds(0, OWN)], acc_hbm.at[pl.ds(off, OWN)])

    @pl.when(w == 31)
    def _():
        pltpu.sync_copy(acc.at[pl.ds(0, 328)], acc_hbm.at[pl.ds(off, 328)])


def _k_edges(src, dst, yw):
    mesh = plsc.VectorSubcoreMesh(core_axis_name="c", subcore_axis_name="s")
    f = pl.kernel(
        _edges_body,
        out_type=jax.ShapeDtypeStruct((N, D), jnp.float32),
        mesh=mesh,
        scratch_types=[pltpu.VMEM((SEG,), jnp.int32),
                       pltpu.VMEM((SEG,), jnp.int32),
                       pltpu.VMEM((LROWS, CH), jnp.int32),
                       pltpu.VMEM((LROWS, CH), jnp.int32),
                       pltpu.VMEM((CH, D), jnp.float32),
                       pltpu.VMEM((ACCR, D), jnp.float32),
                       pltpu.SemaphoreType.DMA],
        compiler_params=pltpu.CompilerParams(needs_layout_passes=False),
        name="egcn_edges",
    )
    return f(src, dst, yw)


def _final_body(acc_ref, yw_ref, dv_ref, out_ref):
    out_ref[...] = dv_ref[...] * (acc_ref[...] + yw_ref[...])


def _k_final(acc, yw, dinv):
    blk = 1000
    return pl.pallas_call(
        _final_body,
        grid=(N // blk,),
        in_specs=[pl.BlockSpec((blk, D), lambda i: (i, 0)),
                  pl.BlockSpec((blk, D), lambda i: (i, 0)),
                  pl.BlockSpec((blk, 1), lambda i: (i, 0))],
        out_specs=pl.BlockSpec((blk, D), lambda i: (i, 0)),
        out_shape=jax.ShapeDtypeStruct((N, D), jnp.float32),
        name="egcn_final",
    )(acc, yw, dinv.reshape(NPAD, 1)[:N])


def kernel(edge_index_mp, emb, weight, p, W_ih, W_hh, b_ih, b_hh):
    src = edge_index_mp[0]
    dst = edge_index_mp[1]

    score, keys, thr = _k_score(emb, p)
    skey, sidx, sval = _k_compact(keys, score, thr)
    rank = _k_rank(skey, sidx)
    xg, topv = _k_gather(sidx, sval, rank, emb)
    w_new = _k_gru(xg, topv, weight, W_ih.T, W_hh.T, b_ih, b_hh)

    deg = _k_deg(dst)
    dinv = _k_dinv(deg).reshape(NPAD)

    xpad = jnp.concatenate(
        [emb, jnp.zeros((NPAD - N, D), jnp.float32)], axis=0)
    yw = _k_xw(xpad, w_new, dinv)

    acc = _k_edges(src, dst, yw)
    return _k_final(acc, yw[:N], dinv)


# speedup vs baseline: 5.6144x; 1.2133x over previous
import functools

import jax
import jax.numpy as jnp
from jax import lax
from jax.experimental import pallas as pl
from jax.experimental.pallas import tpu as pltpu
from jax.experimental.pallas import tpu_sc as plsc

N = 10000
D = 256
E = 160000
NPAD = 10240
K = 256
SURV = 1024
ZROW = 10000
HALF = 5000
HROWS = 5120

_I32_MIN = -2147483648


def _f32_key(x):
    b = lax.bitcast_convert_type(x, jnp.int32)
    return lax.bitwise_xor(b, lax.bitwise_and(lax.shift_right_arithmetic(b, 31),
                                              jnp.int32(0x7FFFFFFF)))


def _score_body(x_ref, p_ref, score_ref, keys_ref, thr_ref):
    p = p_ref[...]
    rn = lax.rsqrt(jnp.sum(p * p))
    score = lax.dot_general(p, x_ref[...],
                            (((1,), (1,)), ((), ())),
                            preferred_element_type=jnp.float32) * rn
    score_ref[...] = score
    key = _f32_key(score)
    keys_ref[...] = key
    ukey = lax.bitcast_convert_type(
        lax.bitwise_xor(key, jnp.int32(_I32_MIN)), jnp.uint32)

    def bit_step(i, t):
        cand = lax.bitwise_or(t, lax.shift_left(jnp.uint32(1),
                                                jnp.uint32(31) - i.astype(jnp.uint32)))
        cnt = jnp.sum((ukey >= cand).astype(jnp.int32))
        return jnp.where(cnt >= K, cand, t)

    t_u = lax.fori_loop(0, 32, bit_step, jnp.uint32(0))
    t_i = lax.bitwise_xor(lax.bitcast_convert_type(t_u, jnp.int32), jnp.int32(_I32_MIN))
    thr_ref[...] = jnp.full((1, 16), t_i, jnp.int32)


def _k_score(x, p):
    return pl.pallas_call(
        _score_body,
        out_shape=(jax.ShapeDtypeStruct((1, N), jnp.float32),
                   jax.ShapeDtypeStruct((1, N), jnp.int32),
                   jax.ShapeDtypeStruct((1, 16), jnp.int32)),
        name="egcn_score",
    )(x, p.reshape(1, D))


def _compact_body(keys_hbm, score_hbm, thr_hbm, skey_hbm, sidx_hbm, sval_hbm,
                  kbuf, sbuf, tbuf, okey, oidx, oval):
    wid = lax.axis_index("s") * 2 + lax.axis_index("c")

    @pl.when(wid == 0)
    def _():
        pltpu.sync_copy(keys_hbm, kbuf)
        pltpu.sync_copy(score_hbm, sbuf)
        pltpu.sync_copy(thr_hbm, tbuf)
        tvec = tbuf[...]
        sent_k = jnp.full((16,), jnp.int32(_I32_MIN), jnp.int32)
        sent_i = jnp.full((16,), jnp.int32(0), jnp.int32)
        sent_v = jnp.full((16,), 0.0, jnp.float32)

        def zero_step(i, _):
            okey[pl.ds(i * 16, 16)] = sent_k
            oidx[pl.ds(i * 16, 16)] = sent_i
            oval[pl.ds(i * 16, 16)] = sent_v
            return 0

        lax.fori_loop(0, SURV // 16, zero_step, 0)
        lane = lax.iota(jnp.int32, 16)

        def step(i, off):
            kv = kbuf[pl.ds(i * 16, 16)]
            sv = sbuf[pl.ds(i * 16, 16)]
            m = kv >= tvec
            pos = off + plsc.cumsum(m.astype(jnp.int32)) - 1
            m = jnp.logical_and(m, pos < SURV)
            plsc.store_scatter(okey, [pos], kv, mask=m)
            plsc.store_scatter(oidx, [pos], i * 16 + lane, mask=m)
            plsc.store_scatter(oval, [pos], sv, mask=m)
            return off + plsc.all_reduce_population_count(m)

        lax.fori_loop(0, N // 16, step, jnp.zeros((16,), jnp.int32))
        pltpu.sync_copy(okey, skey_hbm)
        pltpu.sync_copy(oidx, sidx_hbm)
        pltpu.sync_copy(oval, sval_hbm)


def _k_compact(keys, score, thr):
    mesh = plsc.VectorSubcoreMesh(core_axis_name="c", subcore_axis_name="s")
    f = pl.kernel(
        _compact_body,
        out_type=(jax.ShapeDtypeStruct((SURV,), jnp.int32),
                  jax.ShapeDtypeStruct((SURV,), jnp.int32),
                  jax.ShapeDtypeStruct((SURV,), jnp.float32)),
        mesh=mesh,
        scratch_types=[pltpu.VMEM((N,), jnp.int32),
                       pltpu.VMEM((N,), jnp.float32),
                       pltpu.VMEM((16,), jnp.int32),
                       pltpu.VMEM((SURV,), jnp.int32),
                       pltpu.VMEM((SURV,), jnp.int32),
                       pltpu.VMEM((SURV,), jnp.float32)],
        compiler_params=pltpu.CompilerParams(needs_layout_passes=False),
        name="egcn_compact",
    )
    return f(keys.reshape(N), score.reshape(N), thr.reshape(16))


def _rank_body(kv_ref, iv_ref, ks_ref, is_ref, rank_ref):
    keys = kv_ref[...]
    idxs = iv_ref[...]

    def step(j, acc):
        r = j // 128
        c = lax.rem(j, 128)
        kj = ks_ref[r, c]
        ij = is_ref[r, c]
        gt = (kj > keys).astype(jnp.int32)
        eq = jnp.logical_and(kj == keys, ij < idxs).astype(jnp.int32)
        return acc + gt + eq

    rank_ref[...] = lax.fori_loop(0, SURV, step,
                                  jnp.zeros((8, 128), jnp.int32))


def _k_rank(skey, sidx):
    k2 = skey.reshape(8, 128)
    i2 = sidx.reshape(8, 128)
    return pl.pallas_call(
        _rank_body,
        out_shape=jax.ShapeDtypeStruct((8, 128), jnp.int32),
        in_specs=[pl.BlockSpec((8, 128), lambda: (0, 0)),
                  pl.BlockSpec((8, 128), lambda: (0, 0)),
                  pl.BlockSpec(memory_space=pltpu.SMEM),
                  pl.BlockSpec(memory_space=pltpu.SMEM)],
        name="egcn_rank",
    )(k2, i2, k2, i2)


def _gather_body(sidx_hbm, sval_hbm, rank_hbm, x_hbm, xg_hbm, topv_hbm,
                 ibuf, vbuf, rbuf, topi, topv, rows, sem):
    wid = lax.axis_index("s") * 2 + lax.axis_index("c")

    @pl.when(wid == 0)
    def _():
        pltpu.sync_copy(sidx_hbm, ibuf)
        pltpu.sync_copy(sval_hbm, vbuf)
        pltpu.sync_copy(rank_hbm, rbuf)

        def step(i, _):
            rk = rbuf[pl.ds(i * 16, 16)]
            m = rk < K
            plsc.store_scatter(topi, [rk], ibuf[pl.ds(i * 16, 16)], mask=m)
            plsc.store_scatter(topv, [rk], vbuf[pl.ds(i * 16, 16)], mask=m)
            return 0

        lax.fori_loop(0, SURV // 16, step, 0)
        pltpu.async_copy(x_hbm.at[topi], rows, sem).wait()
        pltpu.sync_copy(rows, xg_hbm)
        pltpu.sync_copy(topv, topv_hbm)


def _k_gather(sidx, sval, rank, x):
    mesh = plsc.VectorSubcoreMesh(core_axis_name="c", subcore_axis_name="s")
    f = pl.kernel(
        _gather_body,
        out_type=(jax.ShapeDtypeStruct((K, D), jnp.float32),
                  jax.ShapeDtypeStruct((K,), jnp.float32)),
        mesh=mesh,
        scratch_types=[pltpu.VMEM((SURV,), jnp.int32),
                       pltpu.VMEM((SURV,), jnp.float32),
                       pltpu.VMEM((SURV,), jnp.int32),
                       pltpu.VMEM((K,), jnp.int32),
                       pltpu.VMEM((K,), jnp.float32),
                       pltpu.VMEM((K, D), jnp.float32),
                       pltpu.SemaphoreType.DMA],
        compiler_params=pltpu.CompilerParams(needs_layout_passes=False),
        name="egcn_gather",
    )
    return f(sidx, sval, rank.reshape(SURV), x)


def _gru_body(xg_ref, tv_ref, w_ref, wih_ref, whh_ref, bih_ref, bhh_ref,
              wnew_ref):
    xt = xg_ref[...] * jnp.tanh(tv_ref[...]).reshape(K, 1)
    gi = jnp.dot(xt, wih_ref[...], preferred_element_type=jnp.float32) \
        + bih_ref[...]
    gh = jnp.dot(w_ref[...], whh_ref[...], preferred_element_type=jnp.float32) \
        + bhh_ref[...]
    i_r, i_z, i_n = gi[:, :D], gi[:, D:2 * D], gi[:, 2 * D:]
    h_r, h_z, h_n = gh[:, :D], gh[:, D:2 * D], gh[:, 2 * D:]
    r = jax.nn.sigmoid(i_r + h_r)
    z = jax.nn.sigmoid(i_z + h_z)
    n = jnp.tanh(i_n + r * h_n)
    wnew_ref[...] = (1.0 - z) * n + z * w_ref[...]


def _k_gru(xg, topv, weight, w_ih_t, w_hh_t, b_ih, b_hh):
    return pl.pallas_call(
        _gru_body,
        out_shape=jax.ShapeDtypeStruct((D, D), jnp.float32),
        name="egcn_gru",
    )(xg, topv.reshape(1, K), weight, w_ih_t, w_hh_t,
      b_ih.reshape(1, 3 * D), b_hh.reshape(1, 3 * D))


EPT = N
DROWS = 79


def _deg_body(dst_hbm, deg_hbm, dbuf, idx2, ones2, zbuf, cbuf, degsp):
    c = lax.axis_index("c")
    s = lax.axis_index("s")

    @pl.when(c == 0)
    def _():
        def zstep(i, _):
            zbuf[pl.ds(i * 16, 16)] = jnp.zeros((16,), jnp.float32)
            return 0

        lax.fori_loop(0, 640 // 16, zstep, 0)
        pltpu.sync_copy(zbuf, degsp.at[pl.ds(s * 640, 640)])
        plsc.subcore_barrier()

        pltpu.sync_copy(dst_hbm.at[pl.ds(s * EPT, EPT)],
                        dbuf.at[pl.ds(0, EPT)])
        lane = lax.iota(jnp.int32, 16)

        def prep(i, _):
            r = i // 8
            col = lax.rem(i, 8) * 16
            pos = i * 16 + lane
            real = pos < EPT
            dv = dbuf[pl.ds(i * 16, 16)]
            dv = jnp.where(real, dv, lax.bitwise_and(pos, jnp.int32(8191)))
            idx2[r, pl.ds(col, 16)] = dv
            ones2[r, pl.ds(col, 16)] = jnp.where(real, 1.0,
                                                 0.0).astype(jnp.float32)
            return 0

        lax.fori_loop(0, DROWS * 8, prep, 0)

        def scat(j, _):
            pltpu.sync_copy(ones2.at[j], degsp.at[idx2.at[j]], add=True)
            return 0

        lax.fori_loop(0, DROWS, scat, 0)
        plsc.subcore_barrier()
        pltpu.sync_copy(degsp.at[pl.ds(s * 640, 640)], cbuf)
        pltpu.sync_copy(cbuf, deg_hbm.at[pl.ds(s * 640, 640)])


def _k_deg(dst):
    mesh = plsc.VectorSubcoreMesh(core_axis_name="c", subcore_axis_name="s")
    f = pl.kernel(
        _deg_body,
        out_type=jax.ShapeDtypeStruct((NPAD,), jnp.float32),
        mesh=mesh,
        scratch_types=[pltpu.VMEM((DROWS * 128,), jnp.int32),
                       pltpu.VMEM((DROWS, 128), jnp.int32),
                       pltpu.VMEM((DROWS, 128), jnp.float32),
                       pltpu.VMEM((640,), jnp.float32),
                       pltpu.VMEM((640,), jnp.float32),
                       pltpu.VMEM_SHARED((NPAD,), jnp.float32)],
        name="egcn_deg",
    )
    return f(dst)


def _dinv_body(deg_ref, dinv_ref):
    row = lax.broadcasted_iota(jnp.int32, (80, 128), 0)
    col = lax.broadcasted_iota(jnp.int32, (80, 128), 1)
    gid = row * 128 + col
    d = lax.rsqrt(deg_ref[...] + 1.0)
    dinv_ref[...] = jnp.where(gid < N, d, 0.0)


def _k_dinv(deg):
    return pl.pallas_call(
        _dinv_body,
        out_shape=jax.ShapeDtypeStruct((80, 128), jnp.float32),
        name="egcn_dinv",
    )(deg.reshape(80, 128))


def _xw_body(x_ref, w_ref, dv_ref, yw_ref):
    acc = jnp.dot(x_ref[...], w_ref[...], preferred_element_type=jnp.float32)
    yw_ref[...] = acc * dv_ref[...]


def _k_xw(xp, w_new, dinv):
    blk = 1024
    return pl.pallas_call(
        _xw_body,
        grid=(NPAD // blk,),
        in_specs=[pl.BlockSpec((blk, D), lambda i: (i, 0)),
                  pl.BlockSpec((D, D), lambda i: (0, 0)),
                  pl.BlockSpec((blk, 1), lambda i: (i, 0))],
        out_specs=pl.BlockSpec((blk, D), lambda i: (i, 0)),
        out_shape=jax.ShapeDtypeStruct((NPAD, D), jnp.float32),
        name="egcn_xw",
    )(xp, w_new, dinv.reshape(NPAD, 1))


OWN = 312
ACCR = 336
CH = 32
SEG = 4000
LCAP = 4160
MAGIC = 107549


def _edges_body(src_hbm, dst_hbm, yw_hbm, acc_hbm,
                sseg, dseg, lsr, llc, rows0, rows1, acc, gsem):
    c = lax.axis_index("c")
    s = lax.axis_index("s")
    w = s * 2 + c
    off_row = w * OWN
    lane = lax.iota(jnp.int32, 16)

    pltpu.sync_copy(yw_hbm.at[pl.ds(ZROW, 128)], acc.at[pl.ds(0, 128)])
    pltpu.sync_copy(yw_hbm.at[pl.ds(ZROW, 128)], acc.at[pl.ds(128, 128)])
    pltpu.sync_copy(yw_hbm.at[pl.ds(ZROW, 80)], acc.at[pl.ds(256, 80)])

    def seg_body(g, _unused):
        pltpu.sync_copy(src_hbm.at[pl.ds(g * SEG, SEG)], sseg)
        pltpu.sync_copy(dst_hbm.at[pl.ds(g * SEG, SEG)], dseg)

        def filt(i, off):
            sv = sseg[pl.ds(i * 16, 16)]
            dv = dseg[pl.ds(i * 16, 16)]
            t = jnp.minimum(lax.shift_right_logical(dv * MAGIC, 25), 31)
            m = t == w
            plsc.store_compressed(lsr.at[pl.ds(off, 16)], sv, mask=m)
            plsc.store_compressed(llc.at[pl.ds(off, 16)], dv - off_row,
                                  mask=m)
            return off + plsc.all_reduce_population_count(m)[0]

        cnt = lax.fori_loop(0, SEG // 16, filt, jnp.int32(0))

        for j in range(2):
            lsr[pl.ds(cnt + j * 16, 16)] = (
                ZROW + lax.bitwise_and(cnt + j * 16 + lane, jnp.int32(127)))
            llc[pl.ds(cnt + j * 16, 16)] = (
                328 + lax.bitwise_and(lane, jnp.int32(7)))

        nch = lax.shift_right_logical(cnt + (CH - 1), 5)

        cp = pltpu.async_copy(yw_hbm.at[lsr.at[pl.ds(0, CH)]], rows0, gsem)

        def adds_from(rbuf, ch):
            for j in range(CH // 16):
                lv = llc[pl.ds(ch * CH + j * 16, 16)]
                for k in range(16):
                    rl = lv[k]
                    e = j * 16 + k
                    for cb in range(D // 16):
                        plsc.addupdate(acc.at[rl, pl.ds(cb * 16, 16)],
                                       rbuf[e, pl.ds(cb * 16, 16)])

        def drain(ch, _):
            even = lax.rem(ch, 2) == 0

            @pl.when(even)
            def _():
                cp.wait()

                @pl.when(ch + 1 < nch)
                def _():
                    pltpu.async_copy(
                        yw_hbm.at[lsr.at[pl.ds((ch + 1) * CH, CH)]],
                        rows1, gsem)

                adds_from(rows0, ch)

            @pl.when(jnp.logical_not(even))
            def _():
                cp.wait()

                @pl.when(ch + 1 < nch)
                def _():
                    pltpu.async_copy(
                        yw_hbm.at[lsr.at[pl.ds((ch + 1) * CH, CH)]],
                        rows0, gsem)

                adds_from(rows1, ch)

            return 0

        lax.fori_loop(0, nch, drain, 0)
        return 0

    lax.fori_loop(0, E // SEG, seg_body, 0)

    @pl.when(w < 31)
    def _():
        pltpu.sync_copy(acc.at[pl.ds(0, OWN)],
                        acc_hbm.at[pl.ds(off_row, OWN)])

    @pl.when(w == 31)
    def _():
        pltpu.sync_copy(acc.at[pl.ds(0, 328)],
                        acc_hbm.at[pl.ds(off_row, 328)])


def _k_edges(src, dst, yw):
    mesh = plsc.VectorSubcoreMesh(core_axis_name="c", subcore_axis_name="s")
    f = pl.kernel(
        _edges_body,
        out_type=jax.ShapeDtypeStruct((N, D), jnp.float32),
        mesh=mesh,
        scratch_types=[pltpu.VMEM((SEG,), jnp.int32),
                       pltpu.VMEM((SEG,), jnp.int32),
                       pltpu.VMEM((LCAP,), jnp.int32),
                       pltpu.VMEM((LCAP,), jnp.int32),
                       pltpu.VMEM((CH, D), jnp.float32),
                       pltpu.VMEM((CH, D), jnp.float32),
                       pltpu.VMEM((ACCR, D), jnp.float32),
                       pltpu.SemaphoreType.DMA],
        compiler_params=pltpu.CompilerParams(needs_layout_passes=False),
        name="egcn_edges",
    )
    return f(src, dst, yw)


def _final_body(acc_ref, yw_ref, dv_ref, out_ref):
    out_ref[...] = dv_ref[...] * (acc_ref[...] + yw_ref[...])


def _k_final(acc, yw, dinv):
    blk = 1000
    return pl.pallas_call(
        _final_body,
        grid=(N // blk,),
        in_specs=[pl.BlockSpec((blk, D), lambda i: (i, 0)),
                  pl.BlockSpec((blk, D), lambda i: (i, 0)),
                  pl.BlockSpec((blk, 1), lambda i: (i, 0))],
        out_specs=pl.BlockSpec((blk, D), lambda i: (i, 0)),
        out_shape=jax.ShapeDtypeStruct((N, D), jnp.float32),
        name="egcn_final",
    )(acc, yw, dinv.reshape(NPAD, 1)[:N])


def kernel(edge_index_mp, emb, weight, p, W_ih, W_hh, b_ih, b_hh):
    src = edge_index_mp[0]
    dst = edge_index_mp[1]

    score, keys, thr = _k_score(emb, p)
    skey, sidx, sval = _k_compact(keys, score, thr)
    rank = _k_rank(skey, sidx)
    xg, topv = _k_gather(sidx, sval, rank, emb)
    w_new = _k_gru(xg, topv, weight, W_ih.T, W_hh.T, b_ih, b_hh)

    deg = _k_deg(dst)
    dinv = _k_dinv(deg).reshape(NPAD)

    xpad = jnp.concatenate(
        [emb, jnp.zeros((NPAD - N, D), jnp.float32)], axis=0)
    yw = _k_xw(xpad, w_new, dinv)

    acc = _k_edges(src, dst, yw)
    return _k_final(acc, yw[:N], dinv)


# speedup vs baseline: 6.3387x; 1.1290x over previous
import functools

import jax
import jax.numpy as jnp
from jax import lax
from jax.experimental import pallas as pl
from jax.experimental.pallas import tpu as pltpu
from jax.experimental.pallas import tpu_sc as plsc

N = 10000
D = 256
E = 160000
NPAD = 10240
K = 256
SURV = 1024
ZROW = 10000
HALF = 5000
HROWS = 5120

_I32_MIN = -2147483648


def _f32_key(x):
    b = lax.bitcast_convert_type(x, jnp.int32)
    return lax.bitwise_xor(b, lax.bitwise_and(lax.shift_right_arithmetic(b, 31),
                                              jnp.int32(0x7FFFFFFF)))


def _score_body(x_ref, p_ref, score_ref, keys_ref, thr_ref):
    p = p_ref[...]
    rn = lax.rsqrt(jnp.sum(p * p))
    score = lax.dot_general(p, x_ref[...],
                            (((1,), (1,)), ((), ())),
                            preferred_element_type=jnp.float32) * rn
    score_ref[...] = score
    key = _f32_key(score)
    keys_ref[...] = key
    ukey = lax.bitcast_convert_type(
        lax.bitwise_xor(key, jnp.int32(_I32_MIN)), jnp.uint32)

    def bit_step(i, t):
        cand = lax.bitwise_or(t, lax.shift_left(jnp.uint32(1),
                                                jnp.uint32(31) - i.astype(jnp.uint32)))
        cnt = jnp.sum((ukey >= cand).astype(jnp.int32))
        return jnp.where(cnt >= K, cand, t)

    t_u = lax.fori_loop(0, 32, bit_step, jnp.uint32(0))
    t_i = lax.bitwise_xor(lax.bitcast_convert_type(t_u, jnp.int32), jnp.int32(_I32_MIN))
    thr_ref[...] = jnp.full((1, 16), t_i, jnp.int32)


def _k_score(x, p):
    return pl.pallas_call(
        _score_body,
        out_shape=(jax.ShapeDtypeStruct((1, N), jnp.float32),
                   jax.ShapeDtypeStruct((1, N), jnp.int32),
                   jax.ShapeDtypeStruct((1, 16), jnp.int32)),
        name="egcn_score",
    )(x, p.reshape(1, D))


def _compact_body(keys_hbm, score_hbm, thr_hbm, skey_hbm, sidx_hbm, sval_hbm,
                  kbuf, sbuf, tbuf, okey, oidx, oval):
    wid = lax.axis_index("s") * 2 + lax.axis_index("c")

    @pl.when(wid == 0)
    def _():
        pltpu.sync_copy(keys_hbm, kbuf)
        pltpu.sync_copy(score_hbm, sbuf)
        pltpu.sync_copy(thr_hbm, tbuf)
        tvec = tbuf[...]
        sent_k = jnp.full((16,), jnp.int32(_I32_MIN), jnp.int32)
        sent_i = jnp.full((16,), jnp.int32(0), jnp.int32)
        sent_v = jnp.full((16,), 0.0, jnp.float32)

        def zero_step(i, _):
            okey[pl.ds(i * 16, 16)] = sent_k
            oidx[pl.ds(i * 16, 16)] = sent_i
            oval[pl.ds(i * 16, 16)] = sent_v
            return 0

        lax.fori_loop(0, SURV // 16, zero_step, 0)
        lane = lax.iota(jnp.int32, 16)

        def step(i, off):
            kv = kbuf[pl.ds(i * 16, 16)]
            sv = sbuf[pl.ds(i * 16, 16)]
            m = kv >= tvec
            pos = off + plsc.cumsum(m.astype(jnp.int32)) - 1
            m = jnp.logical_and(m, pos < SURV)
            plsc.store_scatter(okey, [pos], kv, mask=m)
            plsc.store_scatter(oidx, [pos], i * 16 + lane, mask=m)
            plsc.store_scatter(oval, [pos], sv, mask=m)
            return off + plsc.all_reduce_population_count(m)

        lax.fori_loop(0, N // 16, step, jnp.zeros((16,), jnp.int32))
        pltpu.sync_copy(okey, skey_hbm)
        pltpu.sync_copy(oidx, sidx_hbm)
        pltpu.sync_copy(oval, sval_hbm)


def _k_compact(keys, score, thr):
    mesh = plsc.VectorSubcoreMesh(core_axis_name="c", subcore_axis_name="s")
    f = pl.kernel(
        _compact_body,
        out_type=(jax.ShapeDtypeStruct((SURV,), jnp.int32),
                  jax.ShapeDtypeStruct((SURV,), jnp.int32),
                  jax.ShapeDtypeStruct((SURV,), jnp.float32)),
        mesh=mesh,
        scratch_types=[pltpu.VMEM((N,), jnp.int32),
                       pltpu.VMEM((N,), jnp.float32),
                       pltpu.VMEM((16,), jnp.int32),
                       pltpu.VMEM((SURV,), jnp.int32),
                       pltpu.VMEM((SURV,), jnp.int32),
                       pltpu.VMEM((SURV,), jnp.float32)],
        compiler_params=pltpu.CompilerParams(needs_layout_passes=False),
        name="egcn_compact",
    )
    return f(keys.reshape(N), score.reshape(N), thr.reshape(16))


def _rank_body(kv_ref, iv_ref, ks_ref, is_ref, rank_ref):
    keys = kv_ref[...]
    idxs = iv_ref[...]

    def step(j, acc):
        r = j // 128
        c = lax.rem(j, 128)
        kj = ks_ref[r, c]
        ij = is_ref[r, c]
        gt = (kj > keys).astype(jnp.int32)
        eq = jnp.logical_and(kj == keys, ij < idxs).astype(jnp.int32)
        return acc + gt + eq

    rank_ref[...] = lax.fori_loop(0, SURV, step,
                                  jnp.zeros((8, 128), jnp.int32))


def _k_rank(skey, sidx):
    k2 = skey.reshape(8, 128)
    i2 = sidx.reshape(8, 128)
    return pl.pallas_call(
        _rank_body,
        out_shape=jax.ShapeDtypeStruct((8, 128), jnp.int32),
        in_specs=[pl.BlockSpec((8, 128), lambda: (0, 0)),
                  pl.BlockSpec((8, 128), lambda: (0, 0)),
                  pl.BlockSpec(memory_space=pltpu.SMEM),
                  pl.BlockSpec(memory_space=pltpu.SMEM)],
        name="egcn_rank",
    )(k2, i2, k2, i2)


def _gather_body(sidx_hbm, sval_hbm, rank_hbm, x_hbm, xg_hbm, topv_hbm,
                 ibuf, vbuf, rbuf, topi, topv, rows, sem):
    wid = lax.axis_index("s") * 2 + lax.axis_index("c")

    @pl.when(wid == 0)
    def _():
        pltpu.sync_copy(sidx_hbm, ibuf)
        pltpu.sync_copy(sval_hbm, vbuf)
        pltpu.sync_copy(rank_hbm, rbuf)

        def step(i, _):
            rk = rbuf[pl.ds(i * 16, 16)]
            m = rk < K
            plsc.store_scatter(topi, [rk], ibuf[pl.ds(i * 16, 16)], mask=m)
            plsc.store_scatter(topv, [rk], vbuf[pl.ds(i * 16, 16)], mask=m)
            return 0

        lax.fori_loop(0, SURV // 16, step, 0)
        pltpu.async_copy(x_hbm.at[topi], rows, sem).wait()
        pltpu.sync_copy(rows, xg_hbm)
        pltpu.sync_copy(topv, topv_hbm)


def _k_gather(sidx, sval, rank, x):
    mesh = plsc.VectorSubcoreMesh(core_axis_name="c", subcore_axis_name="s")
    f = pl.kernel(
        _gather_body,
        out_type=(jax.ShapeDtypeStruct((K, D), jnp.float32),
                  jax.ShapeDtypeStruct((K,), jnp.float32)),
        mesh=mesh,
        scratch_types=[pltpu.VMEM((SURV,), jnp.int32),
                       pltpu.VMEM((SURV,), jnp.float32),
                       pltpu.VMEM((SURV,), jnp.int32),
                       pltpu.VMEM((K,), jnp.int32),
                       pltpu.VMEM((K,), jnp.float32),
                       pltpu.VMEM((K, D), jnp.float32),
                       pltpu.SemaphoreType.DMA],
        compiler_params=pltpu.CompilerParams(needs_layout_passes=False),
        name="egcn_gather",
    )
    return f(sidx, sval, rank.reshape(SURV), x)


def _gru_body(xg_ref, tv_ref, w_ref, wih_ref, whh_ref, bih_ref, bhh_ref,
              wnew_ref):
    xt = xg_ref[...] * jnp.tanh(tv_ref[...]).reshape(K, 1)
    gi = jnp.dot(xt, wih_ref[...], preferred_element_type=jnp.float32) \
        + bih_ref[...]
    gh = jnp.dot(w_ref[...], whh_ref[...], preferred_element_type=jnp.float32) \
        + bhh_ref[...]
    i_r, i_z, i_n = gi[:, :D], gi[:, D:2 * D], gi[:, 2 * D:]
    h_r, h_z, h_n = gh[:, :D], gh[:, D:2 * D], gh[:, 2 * D:]
    r = jax.nn.sigmoid(i_r + h_r)
    z = jax.nn.sigmoid(i_z + h_z)
    n = jnp.tanh(i_n + r * h_n)
    wnew_ref[...] = (1.0 - z) * n + z * w_ref[...]


def _k_gru(xg, topv, weight, w_ih_t, w_hh_t, b_ih, b_hh):
    return pl.pallas_call(
        _gru_body,
        out_shape=jax.ShapeDtypeStruct((D, D), jnp.float32),
        name="egcn_gru",
    )(xg, topv.reshape(1, K), weight, w_ih_t, w_hh_t,
      b_ih.reshape(1, 3 * D), b_hh.reshape(1, 3 * D))


EPT = N
DROWS = 79


def _deg_body(dst_hbm, deg_hbm, dbuf, idx2, ones2, zbuf, cbuf, degsp):
    c = lax.axis_index("c")
    s = lax.axis_index("s")

    @pl.when(c == 0)
    def _():
        def zstep(i, _):
            zbuf[pl.ds(i * 16, 16)] = jnp.zeros((16,), jnp.float32)
            return 0

        lax.fori_loop(0, 640 // 16, zstep, 0)
        pltpu.sync_copy(zbuf, degsp.at[pl.ds(s * 640, 640)])
        plsc.subcore_barrier()

        pltpu.sync_copy(dst_hbm.at[pl.ds(s * EPT, EPT)],
                        dbuf.at[pl.ds(0, EPT)])
        lane = lax.iota(jnp.int32, 16)

        def prep(i, _):
            r = i // 8
            col = lax.rem(i, 8) * 16
            pos = i * 16 + lane
            real = pos < EPT
            dv = dbuf[pl.ds(i * 16, 16)]
            dv = jnp.where(real, dv, lax.bitwise_and(pos, jnp.int32(8191)))
            idx2[r, pl.ds(col, 16)] = dv
            ones2[r, pl.ds(col, 16)] = jnp.where(real, 1.0,
                                                 0.0).astype(jnp.float32)
            return 0

        lax.fori_loop(0, DROWS * 8, prep, 0)

        def scat(j, _):
            pltpu.sync_copy(ones2.at[j], degsp.at[idx2.at[j]], add=True)
            return 0

        lax.fori_loop(0, DROWS, scat, 0)
        plsc.subcore_barrier()
        pltpu.sync_copy(degsp.at[pl.ds(s * 640, 640)], cbuf)
        pltpu.sync_copy(cbuf, deg_hbm.at[pl.ds(s * 640, 640)])


def _k_deg(dst):
    mesh = plsc.VectorSubcoreMesh(core_axis_name="c", subcore_axis_name="s")
    f = pl.kernel(
        _deg_body,
        out_type=jax.ShapeDtypeStruct((NPAD,), jnp.float32),
        mesh=mesh,
        scratch_types=[pltpu.VMEM((DROWS * 128,), jnp.int32),
                       pltpu.VMEM((DROWS, 128), jnp.int32),
                       pltpu.VMEM((DROWS, 128), jnp.float32),
                       pltpu.VMEM((640,), jnp.float32),
                       pltpu.VMEM((640,), jnp.float32),
                       pltpu.VMEM_SHARED((NPAD,), jnp.float32)],
        name="egcn_deg",
    )
    return f(dst)


def _dinv_body(deg_ref, dinv_ref):
    row = lax.broadcasted_iota(jnp.int32, (80, 128), 0)
    col = lax.broadcasted_iota(jnp.int32, (80, 128), 1)
    gid = row * 128 + col
    d = lax.rsqrt(deg_ref[...] + 1.0)
    dinv_ref[...] = jnp.where(gid < N, d, 0.0)


def _k_dinv(deg):
    return pl.pallas_call(
        _dinv_body,
        out_shape=jax.ShapeDtypeStruct((80, 128), jnp.float32),
        name="egcn_dinv",
    )(deg.reshape(80, 128))


def _xw_body(x_ref, w_ref, dv_ref, yw_ref):
    acc = jnp.dot(x_ref[...], w_ref[...], preferred_element_type=jnp.float32)
    yw_ref[...] = acc * dv_ref[...]


def _k_xw(xp, w_new, dinv):
    blk = 1024
    return pl.pallas_call(
        _xw_body,
        grid=(NPAD // blk,),
        in_specs=[pl.BlockSpec((blk, D), lambda i: (i, 0)),
                  pl.BlockSpec((D, D), lambda i: (0, 0)),
                  pl.BlockSpec((blk, 1), lambda i: (i, 0))],
        out_specs=pl.BlockSpec((blk, D), lambda i: (i, 0)),
        out_shape=jax.ShapeDtypeStruct((NPAD, D), jnp.float32),
        name="egcn_xw",
    )(xp, w_new, dinv.reshape(NPAD, 1))


OWN = 312
ACCR = 336
CH = 32
RCAP = 5008
LCAP = 5072
MAGIC = 107549
NB = 4


def _owner(dv):
    return jnp.minimum(lax.shift_right_logical(dv * MAGIC, 25), 31)


def _part_body(src_hbm, dst_hbm, psrc_hbm, pdst_hbm, cnt_hbm,
               sbuf, dbuf, bs0, bd0, bs1, bd1, bs2, bd2, bs3, bd3, cbuf):
    c = lax.axis_index("c")
    s = lax.axis_index("s")
    t = s * 2 + c
    lane = lax.iota(jnp.int32, 16)
    ept = E // 32

    pltpu.sync_copy(src_hbm.at[pl.ds(t * ept, ept)], sbuf.at[pl.ds(0, ept)])
    pltpu.sync_copy(dst_hbm.at[pl.ds(t * ept, ept)], dbuf.at[pl.ds(0, ept)])

    bsrc = (bs0, bs1, bs2, bs3)
    bdst = (bd0, bd1, bd2, bd3)

    def filt(i, offs):
        sv = sbuf[pl.ds(i * 16, 16)]
        dv = dbuf[pl.ds(i * 16, 16)]
        cb = lax.shift_right_logical(_owner(dv), 3)
        valid = (i * 16 + lane) < ept
        new_offs = []
        for b in range(NB):
            m = jnp.logical_and(cb == b, valid)
            plsc.store_compressed(bsrc[b].at[pl.ds(offs[b], 16)], sv, mask=m)
            plsc.store_compressed(bdst[b].at[pl.ds(offs[b], 16)], dv, mask=m)
            new_offs.append(offs[b] +
                            plsc.all_reduce_population_count(m)[0])
        return tuple(new_offs)

    offs = lax.fori_loop(0, RCAP // 16, filt,
                         (jnp.int32(0),) * NB)

    for b in range(NB):
        reg = (b * 32) * RCAP + t * RCAP
        pltpu.sync_copy(bsrc[b], psrc_hbm.at[pl.ds(reg, RCAP)])
        pltpu.sync_copy(bdst[b], pdst_hbm.at[pl.ds(reg, RCAP)])

    cvec = jnp.where(lane == 0, offs[0],
                     jnp.where(lane == 1, offs[1],
                               jnp.where(lane == 2, offs[2],
                                         jnp.where(lane == 3, offs[3], 0))))
    cbuf[pl.ds(0, 16)] = cvec
    pltpu.sync_copy(cbuf, cnt_hbm.at[pl.ds(t * 16, 16)])


def _k_part(src, dst):
    mesh = plsc.VectorSubcoreMesh(core_axis_name="c", subcore_axis_name="s")
    f = pl.kernel(
        _part_body,
        out_type=(jax.ShapeDtypeStruct((128 * RCAP,), jnp.int32),
                  jax.ShapeDtypeStruct((128 * RCAP,), jnp.int32),
                  jax.ShapeDtypeStruct((512,), jnp.int32)),
        mesh=mesh,
        scratch_types=[pltpu.VMEM((RCAP,), jnp.int32),
                       pltpu.VMEM((RCAP,), jnp.int32)]
                      + [pltpu.VMEM((RCAP,), jnp.int32)] * 8
                      + [pltpu.VMEM((16,), jnp.int32)],
        compiler_params=pltpu.CompilerParams(needs_layout_passes=False),
        name="egcn_part",
    )
    return f(src, dst)


def _edges_body(psrc_hbm, pdst_hbm, cnt_hbm, yw_hbm, acc_hbm,
                sseg, dseg, lsr, llc, rows0, rows1, acc, cntb, gsem):
    c = lax.axis_index("c")
    s = lax.axis_index("s")
    w = s * 2 + c
    b = lax.shift_right_logical(w, 3)
    off_row = w * OWN
    lane = lax.iota(jnp.int32, 16)

    pltpu.sync_copy(yw_hbm.at[pl.ds(ZROW, 128)], acc.at[pl.ds(0, 128)])
    pltpu.sync_copy(yw_hbm.at[pl.ds(ZROW, 128)], acc.at[pl.ds(128, 128)])
    pltpu.sync_copy(yw_hbm.at[pl.ds(ZROW, 80)], acc.at[pl.ds(256, 80)])
    pltpu.sync_copy(cnt_hbm, cntb)

    def region(r, _unused):
        cnt_r = plsc.load_gather(cntb, [lane * 0 + (r * 16 + b)])[0]
        reg = b * 32 * RCAP + r * RCAP
        pltpu.sync_copy(psrc_hbm.at[pl.ds(reg, RCAP)], sseg)
        pltpu.sync_copy(pdst_hbm.at[pl.ds(reg, RCAP)], dseg)

        def filt(i, off):
            sv = sseg[pl.ds(i * 16, 16)]
            dv = dseg[pl.ds(i * 16, 16)]
            m = jnp.logical_and(_owner(dv) == w, (i * 16 + lane) < cnt_r)
            plsc.store_compressed(lsr.at[pl.ds(off, 16)], sv, mask=m)
            plsc.store_compressed(llc.at[pl.ds(off, 16)], dv - off_row,
                                  mask=m)
            return off + plsc.all_reduce_population_count(m)[0]

        nv = lax.shift_right_logical(cnt_r + 15, 4)
        cnt = lax.fori_loop(0, nv, filt, jnp.int32(0))

        for j in range(2):
            lsr[pl.ds(cnt + j * 16, 16)] = (
                ZROW + lax.bitwise_and(cnt + j * 16 + lane, jnp.int32(127)))
            llc[pl.ds(cnt + j * 16, 16)] = (
                328 + lax.bitwise_and(lane, jnp.int32(7)))

        nch = lax.shift_right_logical(cnt + (CH - 1), 5)

        cp = pltpu.async_copy(yw_hbm.at[lsr.at[pl.ds(0, CH)]], rows0, gsem)

        def adds_from(rbuf, ch):
            for j in range(CH // 16):
                lv = llc[pl.ds(ch * CH + j * 16, 16)]
                for k in range(16):
                    rl = lv[k]
                    e = j * 16 + k
                    for cb2 in range(D // 16):
                        plsc.addupdate(acc.at[rl, pl.ds(cb2 * 16, 16)],
                                       rbuf[e, pl.ds(cb2 * 16, 16)])

        def drain(ch, _):
            even = lax.rem(ch, 2) == 0

            @pl.when(even)
            def _():
                cp.wait()

                @pl.when(ch + 1 < nch)
                def _():
                    pltpu.async_copy(
                        yw_hbm.at[lsr.at[pl.ds((ch + 1) * CH, CH)]],
                        rows1, gsem)

                adds_from(rows0, ch)

            @pl.when(jnp.logical_not(even))
            def _():
                cp.wait()

                @pl.when(ch + 1 < nch)
                def _():
                    pltpu.async_copy(
                        yw_hbm.at[lsr.at[pl.ds((ch + 1) * CH, CH)]],
                        rows0, gsem)

                adds_from(rows1, ch)

            return 0

        lax.fori_loop(0, nch, drain, 0)
        return 0

    lax.fori_loop(0, 32, region, 0)

    @pl.when(w < 31)
    def _():
        pltpu.sync_copy(acc.at[pl.ds(0, OWN)],
                        acc_hbm.at[pl.ds(off_row, OWN)])

    @pl.when(w == 31)
    def _():
        pltpu.sync_copy(acc.at[pl.ds(0, 328)],
                        acc_hbm.at[pl.ds(off_row, 328)])


def _k_edges(psrc, pdst, cnts, yw):
    mesh = plsc.VectorSubcoreMesh(core_axis_name="c", subcore_axis_name="s")
    f = pl.kernel(
        _edges_body,
        out_type=jax.ShapeDtypeStruct((N, D), jnp.float32),
        mesh=mesh,
        scratch_types=[pltpu.VMEM((RCAP,), jnp.int32),
                       pltpu.VMEM((RCAP,), jnp.int32),
                       pltpu.VMEM((LCAP,), jnp.int32),
                       pltpu.VMEM((LCAP,), jnp.int32),
                       pltpu.VMEM((CH, D), jnp.float32),
                       pltpu.VMEM((CH, D), jnp.float32),
                       pltpu.VMEM((ACCR, D), jnp.float32),
                       pltpu.VMEM((512,), jnp.int32),
                       pltpu.SemaphoreType.DMA],
        compiler_params=pltpu.CompilerParams(needs_layout_passes=False),
        name="egcn_edges",
    )
    return f(psrc, pdst, cnts, yw)


def _final_body(acc_ref, yw_ref, dv_ref, out_ref):
    out_ref[...] = dv_ref[...] * (acc_ref[...] + yw_ref[...])


def _k_final(acc, yw, dinv):
    blk = 1000
    return pl.pallas_call(
        _final_body,
        grid=(N // blk,),
        in_specs=[pl.BlockSpec((blk, D), lambda i: (i, 0)),
                  pl.BlockSpec((blk, D), lambda i: (i, 0)),
                  pl.BlockSpec((blk, 1), lambda i: (i, 0))],
        out_specs=pl.BlockSpec((blk, D), lambda i: (i, 0)),
        out_shape=jax.ShapeDtypeStruct((N, D), jnp.float32),
        name="egcn_final",
    )(acc, yw, dinv.reshape(NPAD, 1)[:N])


def kernel(edge_index_mp, emb, weight, p, W_ih, W_hh, b_ih, b_hh):
    src = edge_index_mp[0]
    dst = edge_index_mp[1]

    score, keys, thr = _k_score(emb, p)
    skey, sidx, sval = _k_compact(keys, score, thr)
    rank = _k_rank(skey, sidx)
    xg, topv = _k_gather(sidx, sval, rank, emb)
    w_new = _k_gru(xg, topv, weight, W_ih.T, W_hh.T, b_ih, b_hh)

    deg = _k_deg(dst)
    dinv = _k_dinv(deg).reshape(NPAD)

    xpad = jnp.concatenate(
        [emb, jnp.zeros((NPAD - N, D), jnp.float32)], axis=0)
    yw = _k_xw(xpad, w_new, dinv)

    psrc, pdst, cnts = _k_part(src, dst)
    acc = _k_edges(psrc, pdst, cnts, yw)
    return _k_final(acc, yw[:N], dinv)
